# baseline (speedup 1.0000x reference)
import functools

import jax
import jax.numpy as jnp
from jax import lax
from jax.experimental import pallas as pl
from jax.experimental.pallas import tpu as pltpu
from jax.experimental.pallas import tpu_sc as plsc

D = 768
F = 3072
E = 8
N = 2048
NE = 2 * N
TILE = 768
MAX_TILES = (NE + TILE - 1) // TILE + E - 1
PAD_N = MAX_TILES * TILE

_INV_SQRT2 = 0.7071067811865476


def _router_body(x_ref, wr_ref, wts_ref, dest_ref, te_ref, lt_ref):
    x = x_ref[...]
    lg = lax.dot_general(wr_ref[...], x, (((1,), (1,)), ((), ())),
                         preferred_element_type=jnp.float32)
    m = jnp.max(lg, axis=0, keepdims=True)
    p = jnp.exp(lg - m)
    p = p / jnp.sum(p, axis=0, keepdims=True)
    row = lax.broadcasted_iota(jnp.int32, p.shape, 0)
    m1 = jnp.max(p, axis=0, keepdims=True)
    a1 = jnp.min(jnp.where(p == m1, row, E), axis=0, keepdims=True)
    p2 = jnp.where(row == a1, -jnp.inf, p)
    m2 = jnp.max(p2, axis=0, keepdims=True)
    a2 = jnp.min(jnp.where(p2 == m2, row, E), axis=0, keepdims=True)
    s = m1 + m2
    wts_ref[...] = jnp.concatenate([m1 / s, m2 / s], axis=0)

    en = jnp.concatenate([a1, a2], axis=0)
    mm = jnp.concatenate(
        [(en == e).astype(jnp.float32) for e in range(E)], axis=0)
    ra = lax.broadcasted_iota(jnp.int32, (N, N), 0)
    rb = lax.broadcasted_iota(jnp.int32, (N, N), 1)
    tri = jnp.where(ra < rb, 1.0, 0.0)
    pp = lax.dot_general(mm, tri, (((1,), (0,)), ((), ())),
                         preferred_element_type=jnp.float32)

    tot0, cnt, ranks = [], [], []
    for e in range(E):
        t0 = (pp[2 * e : 2 * e + 1, N - 1 : N]
              + mm[2 * e : 2 * e + 1, N - 1 : N])
        t1 = (pp[2 * e + 1 : 2 * e + 2, N - 1 : N]
              + mm[2 * e + 1 : 2 * e + 2, N - 1 : N])
        tot0.append(t0)
        cnt.append(t0 + t1)
        r0 = pp[2 * e : 2 * e + 1, :]
        r1 = pp[2 * e + 1 : 2 * e + 2, :] + t0
        ranks.append(jnp.concatenate([r0, r1], axis=0))

    nt = [jnp.floor((c + (TILE - 1)) * (1.0 / TILE)) for c in cnt]
    ts_incl = []
    acc = nt[0]
    ts_incl.append(acc)
    for e in range(1, E):
        acc = acc + nt[e]
        ts_incl.append(acc)
    live = ts_incl[E - 1]

    dest = jnp.zeros((2, N), jnp.float32)
    for e in range(E):
        start_e = (ts_incl[e] - nt[e]) * float(TILE)
        dest = dest + mm[2 * e : 2 * e + 2, :] * (ranks[e] + start_e)
    dest_ref[...] = dest.astype(jnp.int32)

    mlane = lax.broadcasted_iota(jnp.int32, (1, 64), 1).astype(jnp.float32)
    mclamp = jnp.minimum(mlane, live - 1.0)
    te = jnp.zeros((1, 64), jnp.float32)
    for e in range(E):
        te = te + jnp.where(mclamp >= ts_incl[e], 1.0, 0.0)
    te_ref[...] = te.astype(jnp.int32)
    l16 = lax.broadcasted_iota(jnp.int32, (1, 16), 1)
    lt_ref[...] = jnp.where(l16 == 0, live.astype(jnp.int32), 0)


def _dispatch_body(dest3, wts3, x, xg, wbuf,
                   d2d, w2d, w16m0, w16m1, rb0, rb1,
                   semS0, semS1, semA, semB):
    wid = lax.axis_index("s") * 2 + lax.axis_index("c")
    lane = lax.iota(jnp.int32, 16)
    tok0 = lax.rem(wid, 16) * 128
    pltpu.sync_copy(dest3.at[wid], d2d)
    pltpu.sync_copy(wts3.at[wid], w2d)
    rbufs = (rb0, rb1)
    wms = (w16m0, w16m1)
    semS = (semS0, semS1)
    stage = [None] * 8
    scatA = [None] * 8
    scatB = [None] * 8
    stage[0] = pltpu.async_copy(x.at[pl.ds(tok0, 16)], rb0, semS0)
    for c in range(8):
        if c >= 1:
            scatA[c - 1].wait()
            scatB[c - 1].wait()
        if c + 1 < 8:
            stage[c + 1] = pltpu.async_copy(
                x.at[pl.ds(tok0 + (c + 1) * 16, 16)],
                rbufs[(c + 1) % 2], semS[(c + 1) % 2])
        stage[c].wait()
        plsc.store_scatter(wms[c % 2], [lane, lane * 0], w2d[c, :])
        idx = d2d.at[c]
        scatA[c] = pltpu.async_copy(rbufs[c % 2], xg.at[idx], semA)
        scatB[c] = pltpu.async_copy(wms[c % 2], wbuf.at[idx], semB)
    scatA[7].wait()
    scatB[7].wait()


def _ffn_body(te_ref, lt_ref, xg_ref, w1_ref, w2_ref, wb_ref, y_ref):
    m = pl.program_id(0)

    @pl.when(m < lt_ref[0])
    def _():
        xt = xg_ref[...]
        h = lax.dot_general(xt, w1_ref[0], (((1,), (1,)), ((), ())),
                            preferred_element_type=jnp.float32)
        h = 0.5 * h * (1.0 + lax.erf(h * _INV_SQRT2))
        y = lax.dot_general(h, w2_ref[0], (((1,), (1,)), ((), ())),
                            preferred_element_type=jnp.float32)
        y_ref[...] = y * wb_ref[:, 0:1]


def _combine_body(y, d03, d13, out,
                  d0m, d1m, rA0, rA1, rB0, rB1, ob0, ob1,
                  semA0, semA1, semB0, semB1, semO0, semO1):
    wid = lax.axis_index("s") * 2 + lax.axis_index("c")
    tok0 = wid * 64
    pltpu.sync_copy(d03.at[wid], d0m)
    pltpu.sync_copy(d13.at[wid], d1m)
    rA = (rA0, rA1)
    rB = (rB0, rB1)
    ob = (ob0, ob1)
    semA = (semA0, semA1)
    semB = (semB0, semB1)
    semO = (semO0, semO1)
    gA = [None] * 4
    gB = [None] * 4
    wo = [None] * 4
    gA[0] = pltpu.async_copy(y.at[d0m.at[0]], rA0, semA0)
    gB[0] = pltpu.async_copy(y.at[d1m.at[0]], rB0, semB0)
    for c in range(4):
        if c >= 2:
            wo[c - 2].wait()
        gA[c].wait()
        gB[c].wait()
        if c + 1 < 4:
            gA[c + 1] = pltpu.async_copy(
                y.at[d0m.at[c + 1]], rA[(c + 1) % 2], semA[(c + 1) % 2])
            gB[c + 1] = pltpu.async_copy(
                y.at[d1m.at[c + 1]], rB[(c + 1) % 2], semB[(c + 1) % 2])
        ra = rA[c % 2]
        rb = rB[c % 2]
        o = ob[c % 2]

        def tok(j, _2, ra=ra, rb=rb, o=o):
            for l in range(D // 16):
                sl = pl.ds(l * 16, 16)
                o[j, sl] = ra[j, sl] + rb[j, sl]
            return 0

        lax.fori_loop(0, 16, tok, 0)
        wo[c] = pltpu.async_copy(
            o, out.at[pl.ds(tok0 + c * 16, 16)], semO[c % 2])
    wo[2].wait()
    wo[3].wait()


def _router_call(x_flat, W_router):
    return pl.pallas_call(
        _router_body,
        out_shape=(jax.ShapeDtypeStruct((2, N), jnp.float32),
                   jax.ShapeDtypeStruct((2, N), jnp.int32),
                   jax.ShapeDtypeStruct((1, 64), jnp.int32),
                   jax.ShapeDtypeStruct((1, 16), jnp.int32)),
    )(x_flat, W_router)


def _ffn_call(te, lt, xg, W1, W2, wbuf):
    grid_spec = pltpu.PrefetchScalarGridSpec(
        num_scalar_prefetch=2,
        grid=(MAX_TILES,),
        in_specs=[
            pl.BlockSpec((TILE, D),
                         lambda m, te, lt: (jnp.minimum(m, lt[0] - 1), 0)),
            pl.BlockSpec((1, F, D),
                         lambda m, te, lt: (te[jnp.minimum(m, lt[0] - 1)], 0, 0)),
            pl.BlockSpec((1, D, F),
                         lambda m, te, lt: (te[jnp.minimum(m, lt[0] - 1)], 0, 0)),
            pl.BlockSpec((TILE, 128),
                         lambda m, te, lt: (jnp.minimum(m, lt[0] - 1), 0)),
        ],
        out_specs=pl.BlockSpec((TILE, D),
                               lambda m, te, lt: (jnp.minimum(m, lt[0] - 1), 0)),
    )
    return pl.pallas_call(
        _ffn_body,
        grid_spec=grid_spec,
        out_shape=jax.ShapeDtypeStruct((PAD_N, D), jnp.float32),
        compiler_params=pltpu.CompilerParams(
            dimension_semantics=("arbitrary",)),
    )(te, lt, xg, W1, W2, wbuf)


@jax.jit
def _moe(x_flat, W_router, W1, W2):
    wts, dest, te, lt = _router_call(x_flat, W_router)

    dispatch = pl.kernel(
        _dispatch_body,
        out_type=(jax.ShapeDtypeStruct((PAD_N, D), jnp.float32),
                  jax.ShapeDtypeStruct((PAD_N, 128), jnp.float32)),
        mesh=plsc.VectorSubcoreMesh(core_axis_name="c", subcore_axis_name="s",
                                    num_cores=2, num_subcores=16),
        scratch_types=[
            pltpu.VMEM((8, 16), jnp.int32),
            pltpu.VMEM((8, 16), jnp.float32),
            pltpu.VMEM((16, 128), jnp.float32),
            pltpu.VMEM((16, 128), jnp.float32),
            pltpu.VMEM((16, D), jnp.float32),
            pltpu.VMEM((16, D), jnp.float32),
            pltpu.SemaphoreType.DMA,
            pltpu.SemaphoreType.DMA,
            pltpu.SemaphoreType.DMA,
            pltpu.SemaphoreType.DMA,
        ],
        compiler_params=pltpu.CompilerParams(needs_layout_passes=False),
    )
    xg, wbuf = dispatch(dest.reshape(32, 8, 16), wts.reshape(32, 8, 16), x_flat)

    y_buf = _ffn_call(te.reshape(64), lt.reshape(16), xg, W1, W2, wbuf)

    combine = pl.kernel(
        _combine_body,
        out_type=jax.ShapeDtypeStruct((N, D), jnp.float32),
        mesh=plsc.VectorSubcoreMesh(core_axis_name="c", subcore_axis_name="s",
                                    num_cores=2, num_subcores=16),
        scratch_types=[
            pltpu.VMEM((4, 16), jnp.int32),
            pltpu.VMEM((4, 16), jnp.int32),
            pltpu.VMEM((16, D), jnp.float32),
            pltpu.VMEM((16, D), jnp.float32),
            pltpu.VMEM((16, D), jnp.float32),
            pltpu.VMEM((16, D), jnp.float32),
            pltpu.VMEM((16, D), jnp.float32),
            pltpu.VMEM((16, D), jnp.float32),
            pltpu.SemaphoreType.DMA,
            pltpu.SemaphoreType.DMA,
            pltpu.SemaphoreType.DMA,
            pltpu.SemaphoreType.DMA,
            pltpu.SemaphoreType.DMA,
            pltpu.SemaphoreType.DMA,
        ],
        compiler_params=pltpu.CompilerParams(needs_layout_passes=False),
    )
    dflat = dest.reshape(NE)
    out = combine(y_buf, dflat[:N].reshape(32, 4, 16), dflat[N:].reshape(32, 4, 16))
    return out


def kernel(x, W_router, W1, W2):
    Bm, Tm, C = x.shape
    x_flat = x.reshape(Bm * Tm, C)
    out = _moe(x_flat, W_router, W1, W2)
    return out.reshape(Bm, Tm, C)


# speedup vs baseline: 1.0586x; 1.0586x over previous
import functools

import jax
import jax.numpy as jnp
from jax import lax
from jax.experimental import pallas as pl
from jax.experimental.pallas import tpu as pltpu
from jax.experimental.pallas import tpu_sc as plsc

D = 768
F = 3072
E = 8
N = 2048
NE = 2 * N
TILE = 640
MAX_TILES = (NE + TILE - 1) // TILE + E - 1
PAD_N = MAX_TILES * TILE

_INV_SQRT2 = 0.7071067811865476


def _router_body(x_ref, wr_ref, wts_ref, dest_ref, te_ref, lt_ref):
    x = x_ref[...]
    lg = lax.dot_general(wr_ref[...], x, (((1,), (1,)), ((), ())),
                         preferred_element_type=jnp.float32)
    m = jnp.max(lg, axis=0, keepdims=True)
    p = jnp.exp(lg - m)
    p = p / jnp.sum(p, axis=0, keepdims=True)
    row = lax.broadcasted_iota(jnp.int32, p.shape, 0)
    m1 = jnp.max(p, axis=0, keepdims=True)
    a1 = jnp.min(jnp.where(p == m1, row, E), axis=0, keepdims=True)
    p2 = jnp.where(row == a1, -jnp.inf, p)
    m2 = jnp.max(p2, axis=0, keepdims=True)
    a2 = jnp.min(jnp.where(p2 == m2, row, E), axis=0, keepdims=True)
    s = m1 + m2
    wts_ref[...] = jnp.concatenate([m1 / s, m2 / s], axis=0)

    en = jnp.concatenate([a1, a2], axis=0)
    mm = jnp.concatenate(
        [(en == e).astype(jnp.float32) for e in range(E)], axis=0)
    ra = lax.broadcasted_iota(jnp.int32, (N, N), 0)
    rb = lax.broadcasted_iota(jnp.int32, (N, N), 1)
    tri = jnp.where(ra < rb, 1.0, 0.0)
    pp = lax.dot_general(mm, tri, (((1,), (0,)), ((), ())),
                         preferred_element_type=jnp.float32)

    tot0, cnt, ranks = [], [], []
    for e in range(E):
        t0 = (pp[2 * e : 2 * e + 1, N - 1 : N]
              + mm[2 * e : 2 * e + 1, N - 1 : N])
        t1 = (pp[2 * e + 1 : 2 * e + 2, N - 1 : N]
              + mm[2 * e + 1 : 2 * e + 2, N - 1 : N])
        tot0.append(t0)
        cnt.append(t0 + t1)
        r0 = pp[2 * e : 2 * e + 1, :]
        r1 = pp[2 * e + 1 : 2 * e + 2, :] + t0
        ranks.append(jnp.concatenate([r0, r1], axis=0))

    nt = [jnp.floor((c + (TILE - 1)) * (1.0 / TILE)) for c in cnt]
    ts_incl = []
    acc = nt[0]
    ts_incl.append(acc)
    for e in range(1, E):
        acc = acc + nt[e]
        ts_incl.append(acc)
    live = ts_incl[E - 1]

    dest = jnp.zeros((2, N), jnp.float32)
    for e in range(E):
        start_e = (ts_incl[e] - nt[e]) * float(TILE)
        dest = dest + mm[2 * e : 2 * e + 2, :] * (ranks[e] + start_e)
    dest_ref[...] = dest.astype(jnp.int32)

    mlane = lax.broadcasted_iota(jnp.int32, (1, 64), 1).astype(jnp.float32)
    mclamp = jnp.minimum(mlane, live - 1.0)
    te = jnp.zeros((1, 64), jnp.float32)
    for e in range(E):
        te = te + jnp.where(mclamp >= ts_incl[e], 1.0, 0.0)
    te_ref[...] = te.astype(jnp.int32)
    l16 = lax.broadcasted_iota(jnp.int32, (1, 16), 1)
    lt_ref[...] = jnp.where(l16 == 0, live.astype(jnp.int32), 0)


def _dispatch_body(dest3, wts3, x, xg, wbuf,
                   d2d, w2d, w16m0, w16m1, rb0, rb1,
                   semS0, semS1, semA, semB):
    wid = lax.axis_index("s") * 2 + lax.axis_index("c")
    lane = lax.iota(jnp.int32, 16)
    tok0 = lax.rem(wid, 16) * 128
    pltpu.sync_copy(dest3.at[wid], d2d)
    pltpu.sync_copy(wts3.at[wid], w2d)
    rbufs = (rb0, rb1)
    wms = (w16m0, w16m1)
    semS = (semS0, semS1)
    stage = [None] * 8
    scatA = [None] * 8
    scatB = [None] * 8
    stage[0] = pltpu.async_copy(x.at[pl.ds(tok0, 16)], rb0, semS0)
    for c in range(8):
        if c >= 1:
            scatA[c - 1].wait()
            scatB[c - 1].wait()
        if c + 1 < 8:
            stage[c + 1] = pltpu.async_copy(
                x.at[pl.ds(tok0 + (c + 1) * 16, 16)],
                rbufs[(c + 1) % 2], semS[(c + 1) % 2])
        stage[c].wait()
        plsc.store_scatter(wms[c % 2], [lane, lane * 0], w2d[c, :])
        idx = d2d.at[c]
        scatA[c] = pltpu.async_copy(rbufs[c % 2], xg.at[idx], semA)
        scatB[c] = pltpu.async_copy(wms[c % 2], wbuf.at[idx], semB)
    scatA[7].wait()
    scatB[7].wait()


def _ffn_body(te_ref, lt_ref, xg_ref, w1_ref, w2_ref, wb_ref, y_ref):
    m = pl.program_id(0)

    @pl.when(m < lt_ref[0])
    def _():
        xt = xg_ref[...]
        h = lax.dot_general(xt, w1_ref[0], (((1,), (1,)), ((), ())),
                            preferred_element_type=jnp.float32)
        h = 0.5 * h * (1.0 + lax.erf(h * _INV_SQRT2))
        y = lax.dot_general(h, w2_ref[0], (((1,), (1,)), ((), ())),
                            preferred_element_type=jnp.float32)
        y_ref[...] = y * wb_ref[:, 0:1]


def _combine_body(y, d03, d13, out,
                  d0m, d1m, rA0, rA1, rB0, rB1, ob0, ob1,
                  semA0, semA1, semB0, semB1, semO0, semO1):
    wid = lax.axis_index("s") * 2 + lax.axis_index("c")
    tok0 = wid * 64
    pltpu.sync_copy(d03.at[wid], d0m)
    pltpu.sync_copy(d13.at[wid], d1m)
    rA = (rA0, rA1)
    rB = (rB0, rB1)
    ob = (ob0, ob1)
    semA = (semA0, semA1)
    semB = (semB0, semB1)
    semO = (semO0, semO1)
    gA = [None] * 4
    gB = [None] * 4
    wo = [None] * 4
    gA[0] = pltpu.async_copy(y.at[d0m.at[0]], rA0, semA0)
    gB[0] = pltpu.async_copy(y.at[d1m.at[0]], rB0, semB0)
    for c in range(4):
        if c >= 2:
            wo[c - 2].wait()
        gA[c].wait()
        gB[c].wait()
        if c + 1 < 4:
            gA[c + 1] = pltpu.async_copy(
                y.at[d0m.at[c + 1]], rA[(c + 1) % 2], semA[(c + 1) % 2])
            gB[c + 1] = pltpu.async_copy(
                y.at[d1m.at[c + 1]], rB[(c + 1) % 2], semB[(c + 1) % 2])
        ra = rA[c % 2]
        rb = rB[c % 2]
        o = ob[c % 2]

        def tok(j, _2, ra=ra, rb=rb, o=o):
            for l in range(D // 16):
                sl = pl.ds(l * 16, 16)
                o[j, sl] = ra[j, sl] + rb[j, sl]
            return 0

        lax.fori_loop(0, 16, tok, 0)
        wo[c] = pltpu.async_copy(
            o, out.at[pl.ds(tok0 + c * 16, 16)], semO[c % 2])
    wo[2].wait()
    wo[3].wait()


def _router_call(x_flat, W_router):
    return pl.pallas_call(
        _router_body,
        out_shape=(jax.ShapeDtypeStruct((2, N), jnp.float32),
                   jax.ShapeDtypeStruct((2, N), jnp.int32),
                   jax.ShapeDtypeStruct((1, 64), jnp.int32),
                   jax.ShapeDtypeStruct((1, 16), jnp.int32)),
    )(x_flat, W_router)


def _ffn_call(te, lt, xg, W1, W2, wbuf):
    grid_spec = pltpu.PrefetchScalarGridSpec(
        num_scalar_prefetch=2,
        grid=(MAX_TILES,),
        in_specs=[
            pl.BlockSpec((TILE, D),
                         lambda m, te, lt: (jnp.minimum(m, lt[0] - 1), 0)),
            pl.BlockSpec((1, F, D),
                         lambda m, te, lt: (te[jnp.minimum(m, lt[0] - 1)], 0, 0)),
            pl.BlockSpec((1, D, F),
                         lambda m, te, lt: (te[jnp.minimum(m, lt[0] - 1)], 0, 0)),
            pl.BlockSpec((TILE, 128),
                         lambda m, te, lt: (jnp.minimum(m, lt[0] - 1), 0)),
        ],
        out_specs=pl.BlockSpec((TILE, D),
                               lambda m, te, lt: (jnp.minimum(m, lt[0] - 1), 0)),
    )
    return pl.pallas_call(
        _ffn_body,
        grid_spec=grid_spec,
        out_shape=jax.ShapeDtypeStruct((PAD_N, D), jnp.float32),
        compiler_params=pltpu.CompilerParams(
            dimension_semantics=("arbitrary",)),
    )(te, lt, xg, W1, W2, wbuf)


@jax.jit
def _moe(x_flat, W_router, W1, W2):
    wts, dest, te, lt = _router_call(x_flat, W_router)

    dispatch = pl.kernel(
        _dispatch_body,
        out_type=(jax.ShapeDtypeStruct((PAD_N, D), jnp.float32),
                  jax.ShapeDtypeStruct((PAD_N, 128), jnp.float32)),
        mesh=plsc.VectorSubcoreMesh(core_axis_name="c", subcore_axis_name="s",
                                    num_cores=2, num_subcores=16),
        scratch_types=[
            pltpu.VMEM((8, 16), jnp.int32),
            pltpu.VMEM((8, 16), jnp.float32),
            pltpu.VMEM((16, 128), jnp.float32),
            pltpu.VMEM((16, 128), jnp.float32),
            pltpu.VMEM((16, D), jnp.float32),
            pltpu.VMEM((16, D), jnp.float32),
            pltpu.SemaphoreType.DMA,
            pltpu.SemaphoreType.DMA,
            pltpu.SemaphoreType.DMA,
            pltpu.SemaphoreType.DMA,
        ],
        compiler_params=pltpu.CompilerParams(needs_layout_passes=False),
    )
    xg, wbuf = dispatch(dest.reshape(32, 8, 16), wts.reshape(32, 8, 16), x_flat)

    y_buf = _ffn_call(te.reshape(64), lt.reshape(16), xg, W1, W2, wbuf)

    combine = pl.kernel(
        _combine_body,
        out_type=jax.ShapeDtypeStruct((N, D), jnp.float32),
        mesh=plsc.VectorSubcoreMesh(core_axis_name="c", subcore_axis_name="s",
                                    num_cores=2, num_subcores=16),
        scratch_types=[
            pltpu.VMEM((4, 16), jnp.int32),
            pltpu.VMEM((4, 16), jnp.int32),
            pltpu.VMEM((16, D), jnp.float32),
            pltpu.VMEM((16, D), jnp.float32),
            pltpu.VMEM((16, D), jnp.float32),
            pltpu.VMEM((16, D), jnp.float32),
            pltpu.VMEM((16, D), jnp.float32),
            pltpu.VMEM((16, D), jnp.float32),
            pltpu.SemaphoreType.DMA,
            pltpu.SemaphoreType.DMA,
            pltpu.SemaphoreType.DMA,
            pltpu.SemaphoreType.DMA,
            pltpu.SemaphoreType.DMA,
            pltpu.SemaphoreType.DMA,
        ],
        compiler_params=pltpu.CompilerParams(needs_layout_passes=False),
    )
    dflat = dest.reshape(NE)
    out = combine(y_buf, dflat[:N].reshape(32, 4, 16), dflat[N:].reshape(32, 4, 16))
    return out


def kernel(x, W_router, W1, W2):
    Bm, Tm, C = x.shape
    x_flat = x.reshape(Bm * Tm, C)
    out = _moe(x_flat, W_router, W1, W2)
    return out.reshape(Bm, Tm, C)
